# Initial kernel scaffold; baseline (speedup 1.0000x reference)
import functools

import jax
import jax.numpy as jnp
from jax.experimental import pallas as pl
from jax.experimental.pallas import tpu as pltpu

_B, _M, _D = 4096, 16384, 256
_TEMP = 0.05
_SHIFT = 1.0 / _TEMP
_BT = 1024
_MT = 1024
_LANES = 128


def _ce_body(x_ref, f_ref, t_ref, out_ref, acc_ref, tacc_ref):
    b = pl.program_id(0)
    m = pl.program_id(1)
    nm = pl.num_programs(1)

    x = x_ref[...]
    ss = jnp.sum(x * x, axis=1, keepdims=True)
    xn = x / jnp.maximum(jnp.sqrt(ss), 1e-12)
    f = f_ref[...]
    s = jax.lax.dot_general(xn, f, (((1,), (1,)), ((), ())),
                            preferred_element_type=jnp.float32)
    s = s / _TEMP
    e = jnp.exp(s - _SHIFT)

    jcol = m * _MT + jax.lax.broadcasted_iota(jnp.int32, (_BT, _MT), 1)
    tcol = t_ref[:, 0:1]
    tm = jnp.where(jcol == tcol, s, 0.0)

    part = e[:, 0:_LANES]
    tpart = tm[:, 0:_LANES]
    for k in range(1, _MT // _LANES):
        part = part + e[:, k * _LANES:(k + 1) * _LANES]
        tpart = tpart + tm[:, k * _LANES:(k + 1) * _LANES]

    @pl.when(m == 0)
    def _():
        acc_ref[...] = part
        tacc_ref[...] = tpart

    @pl.when(m > 0)
    def _():
        acc_ref[...] += part
        tacc_ref[...] += tpart

    @pl.when(m == nm - 1)
    def _():
        lse = _SHIFT + jnp.log(jnp.sum(acc_ref[...], axis=1))
        tgt = jnp.sum(tacc_ref[...], axis=1)
        partial = jnp.sum(lse - tgt) * (1.0 / _B)

        @pl.when(b == 0)
        def _():
            out_ref[0, 0] = partial

        @pl.when(b > 0)
        def _():
            out_ref[0, 0] += partial


def _fused_ce(inputs, t128, cluster_features):
    nb, nm = _B // _BT, _M // _MT
    out = pl.pallas_call(
        _ce_body,
        grid=(nb, nm),
        in_specs=[
            pl.BlockSpec((_BT, _D), lambda b, m: (b, 0)),
            pl.BlockSpec((_MT, _D), lambda b, m: (m, 0)),
            pl.BlockSpec((_BT, _LANES), lambda b, m: (b, 0)),
        ],
        out_specs=pl.BlockSpec((1, 1), lambda b, m: (0, 0)),
        out_shape=jax.ShapeDtypeStruct((1, 1), jnp.float32),
        scratch_shapes=[
            pltpu.VMEM((_BT, _LANES), jnp.float32),
            pltpu.VMEM((_BT, _LANES), jnp.float32),
        ],
        compiler_params=pltpu.CompilerParams(
            dimension_semantics=("arbitrary", "arbitrary"),
        ),
    )(inputs, t128, cluster_features)
    return out[0, 0]


def kernel(inputs, targets, cam_ids, cluster_features):
    t128 = jnp.broadcast_to(targets.astype(jnp.int32)[:, None], (_B, _LANES))
    return _fused_ce(inputs, t128, cluster_features)


# speedup vs baseline: 3.2907x; 3.2907x over previous
import functools

import jax
import jax.numpy as jnp
from jax.experimental import pallas as pl
from jax.experimental.pallas import tpu as pltpu

_B, _M, _D = 4096, 16384, 256
_TEMP = 0.05
_SHIFT = 1.0 / _TEMP
_BT = 1024
_MT = 1024
_LANES = 128


def _ce_body(x_ref, f_ref, t_ref, out_ref, acc_ref, tacc_ref):
    b = pl.program_id(0)
    m = pl.program_id(1)
    nm = pl.num_programs(1)

    x = x_ref[...]
    ss = jnp.sum(x * x, axis=1, keepdims=True)
    xn = x / jnp.maximum(jnp.sqrt(ss), 1e-12)
    f = f_ref[...]
    s = jax.lax.dot_general(xn, f, (((1,), (1,)), ((), ())),
                            preferred_element_type=jnp.float32)
    s = s / _TEMP
    e = jnp.exp(s - _SHIFT)

    jcol = m * _MT + jax.lax.broadcasted_iota(jnp.int32, (_BT, _MT), 1)
    tcol = t_ref[:, 0:1]
    tm = jnp.where(jcol == tcol, s, 0.0)

    part = e[:, 0:_LANES]
    tpart = tm[:, 0:_LANES]
    for k in range(1, _MT // _LANES):
        part = part + e[:, k * _LANES:(k + 1) * _LANES]
        tpart = tpart + tm[:, k * _LANES:(k + 1) * _LANES]

    @pl.when(m == 0)
    def _():
        acc_ref[...] = part
        tacc_ref[...] = tpart

    @pl.when(m > 0)
    def _():
        acc_ref[...] += part
        tacc_ref[...] += tpart

    @pl.when(m == nm - 1)
    def _():
        lse = _SHIFT + jnp.log(jnp.sum(acc_ref[...], axis=1))
        tgt = jnp.sum(tacc_ref[...], axis=1)
        partial = jnp.sum(lse - tgt) * (1.0 / _B)

        @pl.when(b == 0)
        def _():
            out_ref[0, 0] = partial

        @pl.when(b > 0)
        def _():
            out_ref[0, 0] += partial


def _fused_ce(inputs, t128, cluster_features):
    nb, nm = _B // _BT, _M // _MT
    out = pl.pallas_call(
        _ce_body,
        grid=(nb, nm),
        in_specs=[
            pl.BlockSpec((_BT, _D), lambda b, m: (b, 0)),
            pl.BlockSpec((_MT, _D), lambda b, m: (m, 0)),
            pl.BlockSpec((_BT, _LANES), lambda b, m: (b, 0)),
        ],
        out_specs=pl.BlockSpec((1, 1), lambda b, m: (0, 0),
                               memory_space=pltpu.SMEM),
        out_shape=jax.ShapeDtypeStruct((1, 1), jnp.float32),
        scratch_shapes=[
            pltpu.VMEM((_BT, _LANES), jnp.float32),
            pltpu.VMEM((_BT, _LANES), jnp.float32),
        ],
        compiler_params=pltpu.CompilerParams(
            dimension_semantics=("arbitrary", "arbitrary"),
        ),
    )(inputs, cluster_features, t128)
    return out[0, 0]


def kernel(inputs, targets, cam_ids, cluster_features):
    t128 = jnp.broadcast_to(targets.astype(jnp.int32)[:, None], (_B, _LANES))
    return _fused_ce(inputs, t128, cluster_features)
